# Initial kernel scaffold; baseline (speedup 1.0000x reference)
import jax
import jax.numpy as jnp
from jax.experimental import pallas as pl


def kernel(image_features, W1, b1, W2, b2, Wp):
    raise NotImplementedError("write your pallas kernel here")


# speedup vs baseline: 1.0888x; 1.0888x over previous
import jax
import jax.numpy as jnp
from jax.experimental import pallas as pl
from jax.experimental.pallas import tpu as pltpu

_B = 64
_C = 96
_HW = 32
_L = 1024
_TOPK = 512
_PATCH = 16
_CODE = 32
_F32 = jnp.float32
_HI = jax.lax.Precision.HIGHEST


def _main_body(ftT_ref, W1_ref, b1_ref, W2_ref, b2_ref, Wp_ref,
               samp_ref, topk_ref, rem_ref, ss_ref, mask_ref, psc_ref,
               mn_ref, mx_ref):
    b = pl.program_id(0)
    ftT = ftT_ref[0]

    hT = jnp.maximum(
        jax.lax.dot_general(W1_ref[...], ftT, (((1,), (0,)), ((), ()))) + b1_ref[...],
        0.0)
    z_row = jax.lax.dot_general(W2_ref[...], hT, (((1,), (0,)), ((), ()))) + b2_ref[...]
    s_row = jax.nn.sigmoid(z_row)
    s_col = jnp.transpose(s_row)

    gt = (s_row > s_col).astype(_F32)
    eq = s_row == s_col
    ia = jax.lax.broadcasted_iota(jnp.int32, (_L, _L), 0)
    jb = jax.lax.broadcasted_iota(jnp.int32, (_L, _L), 1)
    tie = jnp.where(eq & (jb < ia), 1.0, 0.0)
    rank_col = jnp.sum(gt + tie, axis=1, keepdims=True)
    R = (rank_col == jb.astype(_F32)).astype(_F32)

    ss_row = jax.lax.dot_general(s_row, R, (((1,), (0,)), ((), ())), precision=_HI)
    iota_row = jax.lax.broadcasted_iota(jnp.int32, (1, _L), 1).astype(_F32)
    so_row = jax.lax.dot_general(iota_row, R, (((1,), (0,)), ((), ())), precision=_HI)
    so_int = jnp.floor(so_row + 0.5).astype(jnp.int32)
    topk_ref[0] = so_int[:, :_TOPK]
    rem_ref[0] = so_int[:, _TOPK:]
    ss_ref[0] = ss_row[:, :_TOPK]

    rank_row = jnp.transpose(rank_col)
    mask_ref[0] = (rank_row < float(_TOPK)).astype(_F32)
    psc_ref[0] = s_row

    mean = jnp.mean(ftT, axis=0, keepdims=True)
    var = jnp.mean((ftT - mean) ** 2, axis=0, keepdims=True)
    normT = (ftT - mean) / jnp.sqrt(var + 1e-5)
    Rs = R[:, :_TOPK] * s_col
    G = jax.lax.dot_general(normT, Rs, (((1,), (0,)), ((), ())), precision=_HI)
    samp_ref[0] = jax.lax.dot_general(Wp_ref[...], G, (((1,), (0,)), ((), ())),
                                      precision=_HI)

    bmn = jnp.min(s_row, axis=1, keepdims=True)
    bmx = jnp.max(s_row, axis=1, keepdims=True)

    @pl.when(b == 0)
    def _():
        mn_ref[...] = bmn
        mx_ref[...] = bmx

    @pl.when(b > 0)
    def _():
        mn_ref[...] = jnp.minimum(mn_ref[...], bmn)
        mx_ref[...] = jnp.maximum(mx_ref[...], bmx)


def _maps_body(mask2_ref, psc2_ref, mn_ref, mx_ref, bmap_ref, smap_ref):
    m = mask2_ref[0]
    p = psc2_ref[0]
    mn = mn_ref[...]
    mx = mx_ref[...]
    rng = jnp.maximum(mx - mn, 1e-5)
    ns = (p - mn) / rng

    hp = _HW * _PATCH
    UT = (jax.lax.broadcasted_iota(jnp.int32, (hp, _HW), 0) // _PATCH
          == jax.lax.broadcasted_iota(jnp.int32, (hp, _HW), 1)).astype(_F32)
    U = (jax.lax.broadcasted_iota(jnp.int32, (_HW, hp), 0)
         == jax.lax.broadcasted_iota(jnp.int32, (_HW, hp), 1) // _PATCH).astype(_F32)

    bm = jax.lax.dot_general(UT, m, (((1,), (0,)), ((), ())), precision=_HI)
    bmap_ref[0, 0] = jax.lax.dot_general(bm, U, (((1,), (0,)), ((), ())),
                                         precision=_HI)
    sm = jax.lax.dot_general(UT, ns, (((1,), (0,)), ((), ())), precision=_HI)
    smap_ref[0, 0] = jax.lax.dot_general(sm, U, (((1,), (0,)), ((), ())),
                                         precision=_HI)


def kernel(image_features, W1, b1, W2, b2, Wp):
    x = image_features
    ftT = x.reshape(_B, _C, _L)
    b1r = b1.reshape(_C, 1)
    b2r = b2.reshape(1, 1)

    full = lambda shape: pl.BlockSpec(shape, lambda b: (0,) * len(shape))
    perb = lambda shape: pl.BlockSpec((1,) + shape, lambda b: (b,) + (0,) * len(shape))

    outs = pl.pallas_call(
        _main_body,
        grid=(_B,),
        in_specs=[
            perb((_C, _L)),
            full((_C, _C)),
            full((_C, 1)),
            full((1, _C)),
            full((1, 1)),
            full((_CODE, _C)),
        ],
        out_specs=[
            perb((_CODE, _TOPK)),
            perb((1, _TOPK)),
            perb((1, _TOPK)),
            perb((1, _TOPK)),
            perb((1, _L)),
            perb((1, _L)),
            full((1, 1)),
            full((1, 1)),
        ],
        out_shape=[
            jax.ShapeDtypeStruct((_B, _CODE, _TOPK), _F32),
            jax.ShapeDtypeStruct((_B, 1, _TOPK), jnp.int32),
            jax.ShapeDtypeStruct((_B, 1, _TOPK), jnp.int32),
            jax.ShapeDtypeStruct((_B, 1, _TOPK), _F32),
            jax.ShapeDtypeStruct((_B, 1, _L), _F32),
            jax.ShapeDtypeStruct((_B, 1, _L), _F32),
            jax.ShapeDtypeStruct((1, 1), _F32),
            jax.ShapeDtypeStruct((1, 1), _F32),
        ],
        compiler_params=pltpu.CompilerParams(
            dimension_semantics=("arbitrary",)),
    )(ftT, W1, b1r, W2, b2r, Wp)

    samp, topk3, rem3, ss3, mask3, psc3, mn, mx = outs

    hp = _HW * _PATCH
    bmap, smap = pl.pallas_call(
        _maps_body,
        grid=(_B,),
        in_specs=[
            perb((_HW, _HW)),
            perb((_HW, _HW)),
            full((1, 1)),
            full((1, 1)),
        ],
        out_specs=[
            pl.BlockSpec((1, 1, hp, hp), lambda b: (b, 0, 0, 0)),
            pl.BlockSpec((1, 1, hp, hp), lambda b: (b, 0, 0, 0)),
        ],
        out_shape=[
            jax.ShapeDtypeStruct((_B, 1, hp, hp), _F32),
            jax.ShapeDtypeStruct((_B, 1, hp, hp), _F32),
        ],
        compiler_params=pltpu.CompilerParams(
            dimension_semantics=("arbitrary",)),
    )(mask3.reshape(_B, _HW, _HW), psc3.reshape(_B, _HW, _HW), mn, mx)

    return (samp,
            topk3.reshape(_B, _TOPK),
            rem3.reshape(_B, _TOPK),
            bmap,
            smap,
            mask3.reshape(_B, _L),
            ss3.reshape(_B, _TOPK))


# speedup vs baseline: 1.6340x; 1.5008x over previous
import jax
import jax.numpy as jnp
from jax.experimental import pallas as pl
from jax.experimental.pallas import tpu as pltpu

_B = 64
_C = 96
_HW = 32
_L = 1024
_TOPK = 512
_PATCH = 16
_CODE = 32
_F32 = jnp.float32
_HI = jax.lax.Precision.HIGHEST


def _main_body(ftT_ref, W1_ref, b1_ref, W2_ref, b2_ref, Wp_ref,
               samp_ref, topk_ref, rem_ref, ss_ref, mask_ref, psc_ref,
               mn_ref, mx_ref):
    b = pl.program_id(0)
    ftT = ftT_ref[0]

    hT = jnp.maximum(
        jax.lax.dot_general(W1_ref[...], ftT, (((1,), (0,)), ((), ()))) + b1_ref[...],
        0.0)
    z_row = jax.lax.dot_general(W2_ref[...], hT, (((1,), (0,)), ((), ()))) + b2_ref[...]
    s_row = jax.nn.sigmoid(z_row)
    s_col = jnp.transpose(s_row)

    gt = (s_row > s_col).astype(_F32)
    eq = s_row == s_col
    ia = jax.lax.broadcasted_iota(jnp.int32, (_L, _L), 0)
    jb = jax.lax.broadcasted_iota(jnp.int32, (_L, _L), 1)
    tie = jnp.where(eq & (jb < ia), 1.0, 0.0)
    rank_col = jnp.sum(gt + tie, axis=1, keepdims=True)

    bf = jnp.bfloat16
    rank16 = rank_col.astype(jnp.int16)
    jb16 = jax.lax.broadcasted_iota(jnp.int16, (_L, _L), 1)
    is_p = rank16 == jb16
    Rb = jnp.where(is_p, jnp.ones((), bf), jnp.zeros((), bf))
    s_bf = s_col.astype(bf)
    Rs512 = Rb[:, :_TOPK] * s_bf

    ss_row = jax.lax.dot_general(jnp.ones((1, _L), bf), Rs512,
                                 (((1,), (0,)), ((), ())),
                                 preferred_element_type=_F32)
    ss_ref[0] = ss_row

    iota2 = jax.lax.broadcasted_iota(jnp.int32, (2, _L), 1)
    row_id = jax.lax.broadcasted_iota(jnp.int32, (2, _L), 0)
    qr = jnp.where(row_id == 0, iota2 // 8, iota2 % 8).astype(bf)
    so2 = jax.lax.dot_general(qr, Rb, (((1,), (0,)), ((), ())),
                              preferred_element_type=_F32)
    so_row = so2[0:1] * 8.0 + so2[1:2]
    so_int = jnp.floor(so_row + 0.5).astype(jnp.int32)
    topk_ref[0] = so_int[:, :_TOPK]
    rem_ref[0] = so_int[:, _TOPK:]

    rank_row = jnp.transpose(rank_col)
    mask_ref[0] = (rank_row < float(_TOPK)).astype(_F32)
    psc_ref[0] = s_row

    mean = jnp.mean(ftT, axis=0, keepdims=True)
    var = jnp.mean((ftT - mean) ** 2, axis=0, keepdims=True)
    normT = (ftT - mean) / jnp.sqrt(var + 1e-5)
    G = jax.lax.dot_general(normT.astype(bf), Rs512, (((1,), (0,)), ((), ())),
                            preferred_element_type=_F32)
    samp_ref[0] = jax.lax.dot_general(Wp_ref[...], G, (((1,), (0,)), ((), ())),
                                      precision=_HI)

    bmn = jnp.min(s_row, axis=1, keepdims=True)
    bmx = jnp.max(s_row, axis=1, keepdims=True)

    @pl.when(b == 0)
    def _():
        mn_ref[...] = bmn
        mx_ref[...] = bmx

    @pl.when(b > 0)
    def _():
        mn_ref[...] = jnp.minimum(mn_ref[...], bmn)
        mx_ref[...] = jnp.maximum(mx_ref[...], bmx)


def _maps_body(mask2_ref, psc2_ref, mn_ref, mx_ref, bmap_ref, smap_ref):
    m = mask2_ref[0]
    p = psc2_ref[0]
    mn = mn_ref[...]
    mx = mx_ref[...]
    rng = jnp.maximum(mx - mn, 1e-5)
    ns = (p - mn) / rng

    hp = _HW * _PATCH
    UT = (jax.lax.broadcasted_iota(jnp.int32, (hp, _HW), 0) // _PATCH
          == jax.lax.broadcasted_iota(jnp.int32, (hp, _HW), 1)).astype(_F32)
    U = (jax.lax.broadcasted_iota(jnp.int32, (_HW, hp), 0)
         == jax.lax.broadcasted_iota(jnp.int32, (_HW, hp), 1) // _PATCH).astype(_F32)

    bm = jax.lax.dot_general(UT, m, (((1,), (0,)), ((), ())), precision=_HI)
    bmap_ref[0, 0] = jax.lax.dot_general(bm, U, (((1,), (0,)), ((), ())),
                                         precision=_HI)
    sm = jax.lax.dot_general(UT, ns, (((1,), (0,)), ((), ())), precision=_HI)
    smap_ref[0, 0] = jax.lax.dot_general(sm, U, (((1,), (0,)), ((), ())),
                                         precision=_HI)


def kernel(image_features, W1, b1, W2, b2, Wp):
    x = image_features
    ftT = x.reshape(_B, _C, _L)
    b1r = b1.reshape(_C, 1)
    b2r = b2.reshape(1, 1)

    full = lambda shape: pl.BlockSpec(shape, lambda b: (0,) * len(shape))
    perb = lambda shape: pl.BlockSpec((1,) + shape, lambda b: (b,) + (0,) * len(shape))

    outs = pl.pallas_call(
        _main_body,
        grid=(_B,),
        in_specs=[
            perb((_C, _L)),
            full((_C, _C)),
            full((_C, 1)),
            full((1, _C)),
            full((1, 1)),
            full((_CODE, _C)),
        ],
        out_specs=[
            perb((_CODE, _TOPK)),
            perb((1, _TOPK)),
            perb((1, _TOPK)),
            perb((1, _TOPK)),
            perb((1, _L)),
            perb((1, _L)),
            full((1, 1)),
            full((1, 1)),
        ],
        out_shape=[
            jax.ShapeDtypeStruct((_B, _CODE, _TOPK), _F32),
            jax.ShapeDtypeStruct((_B, 1, _TOPK), jnp.int32),
            jax.ShapeDtypeStruct((_B, 1, _TOPK), jnp.int32),
            jax.ShapeDtypeStruct((_B, 1, _TOPK), _F32),
            jax.ShapeDtypeStruct((_B, 1, _L), _F32),
            jax.ShapeDtypeStruct((_B, 1, _L), _F32),
            jax.ShapeDtypeStruct((1, 1), _F32),
            jax.ShapeDtypeStruct((1, 1), _F32),
        ],
        compiler_params=pltpu.CompilerParams(
            dimension_semantics=("arbitrary",)),
    )(ftT, W1, b1r, W2, b2r, Wp)

    samp, topk3, rem3, ss3, mask3, psc3, mn, mx = outs

    hp = _HW * _PATCH
    bmap, smap = pl.pallas_call(
        _maps_body,
        grid=(_B,),
        in_specs=[
            perb((_HW, _HW)),
            perb((_HW, _HW)),
            full((1, 1)),
            full((1, 1)),
        ],
        out_specs=[
            pl.BlockSpec((1, 1, hp, hp), lambda b: (b, 0, 0, 0)),
            pl.BlockSpec((1, 1, hp, hp), lambda b: (b, 0, 0, 0)),
        ],
        out_shape=[
            jax.ShapeDtypeStruct((_B, 1, hp, hp), _F32),
            jax.ShapeDtypeStruct((_B, 1, hp, hp), _F32),
        ],
        compiler_params=pltpu.CompilerParams(
            dimension_semantics=("arbitrary",)),
    )(mask3.reshape(_B, _HW, _HW), psc3.reshape(_B, _HW, _HW), mn, mx)

    return (samp,
            topk3.reshape(_B, _TOPK),
            rem3.reshape(_B, _TOPK),
            bmap,
            smap,
            mask3.reshape(_B, _L),
            ss3.reshape(_B, _TOPK))
